# baseline (speedup 1.0000x reference)
import functools

import jax
import jax.numpy as jnp
from jax import lax
from jax.experimental import pallas as pl
from jax.experimental.pallas import tpu as pltpu
from jax.experimental.pallas import tpu_sc as plsc

_DIM = 32
_BATCH = 16384
_A = 1.0
_B = 1.0

_info = plsc.get_sparse_core_info()
_NC, _NS, _L = _info.num_cores, _info.num_subcores, _info.num_lanes
_NW = _NC * _NS
_BPW = _BATCH // _NW
_NGRP = _BPW // _L

_mesh = plsc.VectorSubcoreMesh(core_axis_name="c", subcore_axis_name="s")


@functools.partial(
    pl.kernel,
    mesh=_mesh,
    compiler_params=pltpu.CompilerParams(
        needs_layout_passes=False, use_tc_tiling_on_sc=False
    ),
    out_type=[
        jax.ShapeDtypeStruct((_NW, _L), jnp.float32),
        jax.ShapeDtypeStruct((_NW, _L), jnp.float32),
    ],
    scratch_types=[
        pltpu.VMEM((_BPW,), jnp.int32),
        pltpu.VMEM((_BPW,), jnp.int32),
        pltpu.VMEM((_BPW,), jnp.float32),
        pltpu.VMEM((_BPW, _DIM), jnp.float32),
        pltpu.VMEM((_BPW, _DIM), jnp.float32),
        pltpu.VMEM((_BPW,), jnp.float32),
        pltpu.VMEM((_L,), jnp.float32),
        pltpu.VMEM((_L,), jnp.float32),
        pltpu.SemaphoreType.DMA,
        pltpu.SemaphoreType.DMA,
        pltpu.SemaphoreType.DMA,
    ],
)
def _wrmf_sc(uid_hbm, iid_hbm, lab_hbm, ut_hbm, it_hbm, bt_hbm,
             loss_out, l2_out,
             uid_v, iid_v, lab_v, urows, irows, bias_v,
             loss_st, l2_st, sem_u, sem_i, sem_b):
    wid = lax.axis_index("s") * _NC + lax.axis_index("c")
    base = wid * _BPW

    pltpu.sync_copy(uid_hbm.at[pl.ds(base, _BPW)], uid_v)
    pltpu.sync_copy(iid_hbm.at[pl.ds(base, _BPW)], iid_v)
    pltpu.sync_copy(lab_hbm.at[pl.ds(base, _BPW)], lab_v)

    cp_u = pltpu.async_copy(ut_hbm.at[uid_v], urows, sem_u)
    cp_i = pltpu.async_copy(it_hbm.at[iid_v], irows, sem_i)
    cp_b = pltpu.async_copy(bt_hbm.at[iid_v], bias_v, sem_b)
    cp_u.wait()
    cp_i.wait()
    cp_b.wait()

    lane = lax.broadcasted_iota(jnp.int32, (_L,), 0)

    def body(g, carry):
        loss_acc, l2_acc = carry
        ridx = g * _L + lane
        acc = jnp.zeros((_L,), jnp.float32)
        sq = jnp.zeros((_L,), jnp.float32)
        for d in range(_DIM):
            didx = jnp.full((_L,), d, jnp.int32)
            uu = plsc.load_gather(urows, [ridx, didx])
            ii = plsc.load_gather(irows, [ridx, didx])
            acc = acc + uu * ii
            sq = sq + (uu * uu + ii * ii)
        lab = lab_v[pl.ds(g * _L, _L)]
        bias = bias_v[pl.ds(g * _L, _L)]
        pred = acc + bias
        w = (_A - _B) * lab + _B
        err = lab - pred
        return loss_acc + w * err * err, l2_acc + sq

    loss_vec, l2_vec = lax.fori_loop(
        0, _NGRP,
        body,
        (jnp.zeros((_L,), jnp.float32), jnp.zeros((_L,), jnp.float32)),
    )

    loss_st[...] = loss_vec
    l2_st[...] = 0.5 * l2_vec
    pltpu.sync_copy(loss_st, loss_out.at[wid])
    pltpu.sync_copy(l2_st, l2_out.at[wid])


def kernel(user_id, item_id, label, user_table, item_table, item_bias_table):
    loss_p, l2_p = _wrmf_sc(
        user_id.astype(jnp.int32),
        item_id.astype(jnp.int32),
        label,
        user_table,
        item_table,
        item_bias_table.reshape(-1),
    )
    return jnp.sum(loss_p), jnp.sum(l2_p)
